# baseline (device time: 59104 ns/iter reference)
import jax
import jax.numpy as jnp
from jax import lax
from jax.experimental import pallas as pl
from jax.experimental.pallas import tpu as pltpu

N_Z = 4
CH = 4
NSLOT = 5
MESH = pl.DeviceIdType.MESH


def kernel(ids, E):
    T = ids.shape[0]
    V_loc, D = E.shape
    Tq = T // 4
    Tc = Tq // CH

    x = lax.axis_index("x")
    y = lax.axis_index("y")
    z = lax.axis_index("z")
    q = 2 * x + y

    ids_q = lax.dynamic_slice(ids, (q * Tq,), (Tq,))
    local = ids_q - z * V_loc
    mask = (local >= 0) & (local < V_loc)
    safe = jnp.where(mask, local, 0).astype(jnp.int32)
    maskf = mask.astype(jnp.float32)[:, None]

    m2 = mask.reshape(CH, Tc)
    order = jnp.argsort(~m2, axis=1).astype(jnp.int32)
    pos_arr = order.reshape(-1)
    idx_arr = jnp.take_along_axis(safe.reshape(CH, Tc), order, axis=1).reshape(-1)
    cnt_arr = m2.sum(axis=1).astype(jnp.int32)

    def body(
        pos_ref,
        idx_ref,
        cnt_ref,
        E_ref,
        mask_ref,
        out_ref,
        gbuf,
        gsem,
        zbuf,
        qbuf,
        zsend,
        zrecv,
        ysend,
        yrecv,
        xsend,
        xrecv,
    ):
        my_x = lax.axis_index("x")
        my_y = lax.axis_index("y")
        my_z = lax.axis_index("z")
        y_peer = (my_x, 1 - my_y, my_z)
        x_peer = (1 - my_x, my_y, my_z)

        def gather_issue(c):
            def lp(t, acc):
                p = pos_ref[c * Tc + t]
                idx = idx_ref[c * Tc + t]
                pltpu.make_async_copy(
                    E_ref.at[idx], gbuf.at[c, p], gsem.at[c]
                ).start()
                return acc

            lax.fori_loop(0, cnt_ref[c], lp, 0)

        def gather_finish(c):
            def lw(t, acc):
                pltpu.make_async_copy(
                    E_ref.at[0], gbuf.at[c, 0], gsem.at[c]
                ).wait()
                return acc

            lax.fori_loop(0, cnt_ref[c], lw, 0)
            zbuf[c, 0, :, :] = jnp.where(
                mask_ref[c * Tc : (c + 1) * Tc, :] > 0,
                gbuf[c, :, :].astype(jnp.bfloat16),
                jnp.bfloat16(0),
            )

        barrier_sem = pltpu.get_barrier_semaphore()
        for dev in (
            (my_x, my_y, lax.rem(my_z + 1, N_Z)),
            (my_x, my_y, lax.rem(my_z + N_Z - 1, N_Z)),
            y_peer,
            x_peer,
        ):
            pl.semaphore_signal(
                barrier_sem, inc=1, device_id=dev, device_id_type=MESH
            )

        def zcopy(c, src_slot, dst_slot, dirn, dev):
            return pltpu.make_async_remote_copy(
                src_ref=zbuf.at[c, src_slot],
                dst_ref=zbuf.at[c, dst_slot],
                send_sem=zsend.at[c, dirn],
                recv_sem=zrecv.at[c, dirn],
                device_id=dev,
                device_id_type=MESH,
            )

        def z_edge(c):
            @pl.when(my_z == 0)
            def _():
                zcopy(c, 0, 1, 0, (my_x, my_y, my_z + 1)).start()

            @pl.when(my_z == N_Z - 1)
            def _():
                zcopy(c, 0, 2, 1, (my_x, my_y, my_z - 1)).start()

        def z_mid(c):
            def rfwd():
                zcopy(c, 0, 1, 0, (my_x, my_y, my_z)).wait_recv()
                zbuf[c, 3, :, :] = zbuf[c, 0, :, :] + zbuf[c, 1, :, :]
                zcopy(c, 3, 1, 0, (my_x, my_y, my_z + 1)).start()

            def lfwd():
                zcopy(c, 0, 2, 1, (my_x, my_y, my_z)).wait_recv()
                zbuf[c, 4, :, :] = zbuf[c, 0, :, :] + zbuf[c, 2, :, :]
                zcopy(c, 4, 2, 1, (my_x, my_y, my_z - 1)).start()

            @pl.when(my_z == 1)
            def _():
                rfwd()
                lfwd()

            @pl.when(my_z == 2)
            def _():
                lfwd()
                rfwd()

        def qcopy(c, src_slot, dst_slot, sems_s, sems_r, si, peer):
            return pltpu.make_async_remote_copy(
                src_ref=qbuf.at[c, src_slot],
                dst_ref=qbuf.at[c, dst_slot],
                send_sem=sems_s.at[c, si],
                recv_sem=sems_r.at[c, si],
                device_id=peer,
                device_id_type=MESH,
            )

        quarters = (
            2 * my_x + my_y,
            2 * my_x + 1 - my_y,
            2 * (1 - my_x) + my_y,
            2 * (1 - my_x) + 1 - my_y,
        )

        def out_piece(c, k):
            out_ref[pl.ds(quarters[k] * Tq + c * Tc, Tc), :] = qbuf[
                c, k, :, :
            ].astype(jnp.float32)

        def tail_start(c):
            @pl.when(my_z == 0)
            def _():
                zcopy(c, 0, 2, 1, (my_x, my_y, my_z)).wait_recv()
                qbuf[c, 0, :, :] = zbuf[c, 0, :, :] + zbuf[c, 2, :, :]

            @pl.when(my_z == N_Z - 1)
            def _():
                zcopy(c, 0, 1, 0, (my_x, my_y, my_z)).wait_recv()
                qbuf[c, 0, :, :] = zbuf[c, 0, :, :] + zbuf[c, 1, :, :]

            @pl.when((my_z > 0) & (my_z < N_Z - 1))
            def _():
                qbuf[c, 0, :, :] = (
                    zbuf[c, 0, :, :] + zbuf[c, 1, :, :] + zbuf[c, 2, :, :]
                )

            out_piece(c, 0)
            qcopy(c, 0, 1, ysend, yrecv, 0, y_peer).start()
            qcopy(c, 0, 2, xsend, xrecv, 0, x_peer).start()

            @pl.when(my_z == 0)
            def _():
                zcopy(c, 0, 1, 0, (my_x, my_y, my_z)).wait_send()

            @pl.when((my_z > 0) & (my_z < N_Z - 1))
            def _():
                zcopy(c, 3, 1, 0, (my_x, my_y, my_z)).wait_send()
                zcopy(c, 4, 2, 1, (my_x, my_y, my_z)).wait_send()

            @pl.when(my_z == N_Z - 1)
            def _():
                zcopy(c, 0, 2, 1, (my_x, my_y, my_z)).wait_send()

        def diag_forward(c):
            if c % 2 == 0:
                qcopy(c, 0, 1, ysend, yrecv, 0, y_peer).wait_recv()
                qcopy(c, 1, 3, xsend, xrecv, 1, x_peer).start()
                out_piece(c, 1)
            else:
                qcopy(c, 0, 2, xsend, xrecv, 0, x_peer).wait_recv()
                qcopy(c, 2, 3, ysend, yrecv, 1, y_peer).start()
                out_piece(c, 2)

        def tail_finish(c):
            if c % 2 == 0:
                qcopy(c, 0, 2, xsend, xrecv, 0, x_peer).wait_recv()
                out_piece(c, 2)
                qcopy(c, 1, 3, xsend, xrecv, 1, x_peer).wait_recv()
                out_piece(c, 3)
                qcopy(c, 0, 0, ysend, yrecv, 0, y_peer).wait_send()
                qcopy(c, 0, 0, xsend, xrecv, 0, x_peer).wait_send()
                qcopy(c, 1, 0, xsend, xrecv, 1, x_peer).wait_send()
            else:
                qcopy(c, 0, 1, ysend, yrecv, 0, y_peer).wait_recv()
                out_piece(c, 1)
                qcopy(c, 2, 3, ysend, yrecv, 1, y_peer).wait_recv()
                out_piece(c, 3)
                qcopy(c, 0, 0, ysend, yrecv, 0, y_peer).wait_send()
                qcopy(c, 0, 0, xsend, xrecv, 0, x_peer).wait_send()
                qcopy(c, 2, 0, ysend, yrecv, 1, y_peer).wait_send()

        gather_issue(0)
        pl.semaphore_wait(barrier_sem, 4)
        gather_finish(0)
        z_edge(0)
        gather_issue(1)
        gather_finish(1)
        z_edge(1)
        z_mid(0)
        gather_issue(2)
        gather_finish(2)
        z_edge(2)
        z_mid(1)
        gather_issue(3)
        gather_finish(3)
        z_edge(3)
        z_mid(2)
        tail_start(0)
        z_mid(3)
        tail_start(1)
        diag_forward(0)
        tail_start(2)
        diag_forward(1)
        tail_start(3)
        diag_forward(2)
        diag_forward(3)
        tail_finish(0)
        tail_finish(1)
        tail_finish(2)
        tail_finish(3)

    grid_spec = pltpu.PrefetchScalarGridSpec(
        num_scalar_prefetch=3,
        in_specs=[
            pl.BlockSpec(memory_space=pl.ANY),
            pl.BlockSpec(memory_space=pltpu.VMEM),
        ],
        out_specs=pl.BlockSpec(memory_space=pltpu.VMEM),
        scratch_shapes=[
            pltpu.VMEM((CH, Tc, D), jnp.float32),
            pltpu.SemaphoreType.DMA((CH,)),
            pltpu.VMEM((CH, NSLOT, Tc, D), jnp.bfloat16),
            pltpu.VMEM((CH, 4, Tc, D), jnp.bfloat16),
            pltpu.SemaphoreType.DMA((CH, 2)),
            pltpu.SemaphoreType.DMA((CH, 2)),
            pltpu.SemaphoreType.DMA((CH, 2)),
            pltpu.SemaphoreType.DMA((CH, 2)),
            pltpu.SemaphoreType.DMA((CH, 2)),
            pltpu.SemaphoreType.DMA((CH, 2)),
        ],
    )

    return pl.pallas_call(
        body,
        out_shape=jax.ShapeDtypeStruct((T, D), jnp.float32),
        grid_spec=grid_spec,
        compiler_params=pltpu.CompilerParams(collective_id=0),
    )(pos_arr, idx_arr, cnt_arr, E, maskf)


# device time: 54417 ns/iter; 1.0861x vs baseline; 1.0861x over previous
import jax
import jax.numpy as jnp
from jax import lax
from jax.experimental import pallas as pl
from jax.experimental.pallas import tpu as pltpu

N_Z = 4
CH = 4
NSLOT = 5
MESH = pl.DeviceIdType.MESH


def kernel(ids, E):
    T = ids.shape[0]
    V_loc, D = E.shape
    Tq = T // 4
    Tc = Tq // CH

    ids32 = ids.astype(jnp.int32)
    ids2d = ids32[:, None]

    def body(
        ids_s_ref,
        E_ref,
        idv_ref,
        out_ref,
        gbuf,
        gsem,
        zbuf,
        qbuf,
        zsend,
        zrecv,
        ysend,
        yrecv,
        xsend,
        xrecv,
    ):
        my_x = lax.axis_index("x")
        my_y = lax.axis_index("y")
        my_z = lax.axis_index("z")
        y_peer = (my_x, 1 - my_y, my_z)
        x_peer = (1 - my_x, my_y, my_z)
        my_q = 2 * my_x + my_y
        vlo = my_z * V_loc

        def gather_issue(c):
            base = my_q * Tq + c * Tc

            def lp(t, acc):
                loc = ids_s_ref[base + t] - vlo
                valid = (loc >= 0) & (loc < V_loc)

                @pl.when(valid)
                def _():
                    pltpu.make_async_copy(
                        E_ref.at[loc], gbuf.at[c, t], gsem.at[c]
                    ).start()

                return acc + valid.astype(jnp.int32)

            return lax.fori_loop(0, Tc, lp, jnp.int32(0), unroll=8)

        def gather_finish(c, cnt):
            def lw(t, acc):
                pltpu.make_async_copy(
                    E_ref.at[0], gbuf.at[c, 0], gsem.at[c]
                ).wait()
                return acc

            lax.fori_loop(0, cnt, lw, 0)
            idv = idv_ref[pl.ds(my_q * Tq + c * Tc, Tc), :]
            m = (idv >= vlo) & (idv < vlo + V_loc)
            zbuf[c, 0, :, :] = jnp.where(
                m, gbuf[c, :, :].astype(jnp.bfloat16), jnp.bfloat16(0)
            )

        barrier_sem = pltpu.get_barrier_semaphore()
        for dev in (
            (my_x, my_y, lax.rem(my_z + 1, N_Z)),
            (my_x, my_y, lax.rem(my_z + N_Z - 1, N_Z)),
            y_peer,
            x_peer,
        ):
            pl.semaphore_signal(
                barrier_sem, inc=1, device_id=dev, device_id_type=MESH
            )

        def zcopy(c, src_slot, dst_slot, dirn, dev):
            return pltpu.make_async_remote_copy(
                src_ref=zbuf.at[c, src_slot],
                dst_ref=zbuf.at[c, dst_slot],
                send_sem=zsend.at[c, dirn],
                recv_sem=zrecv.at[c, dirn],
                device_id=dev,
                device_id_type=MESH,
            )

        def z_edge(c):
            @pl.when(my_z == 0)
            def _():
                zcopy(c, 0, 1, 0, (my_x, my_y, my_z + 1)).start()

            @pl.when(my_z == N_Z - 1)
            def _():
                zcopy(c, 0, 2, 1, (my_x, my_y, my_z - 1)).start()

        def z_mid(c):
            def rfwd():
                zcopy(c, 0, 1, 0, (my_x, my_y, my_z)).wait_recv()
                zbuf[c, 3, :, :] = zbuf[c, 0, :, :] + zbuf[c, 1, :, :]
                zcopy(c, 3, 1, 0, (my_x, my_y, my_z + 1)).start()

            def lfwd():
                zcopy(c, 0, 2, 1, (my_x, my_y, my_z)).wait_recv()
                zbuf[c, 4, :, :] = zbuf[c, 0, :, :] + zbuf[c, 2, :, :]
                zcopy(c, 4, 2, 1, (my_x, my_y, my_z - 1)).start()

            @pl.when(my_z == 1)
            def _():
                rfwd()
                lfwd()

            @pl.when(my_z == 2)
            def _():
                lfwd()
                rfwd()

        def qcopy(c, src_slot, dst_slot, sems_s, sems_r, si, peer):
            return pltpu.make_async_remote_copy(
                src_ref=qbuf.at[c, src_slot],
                dst_ref=qbuf.at[c, dst_slot],
                send_sem=sems_s.at[c, si],
                recv_sem=sems_r.at[c, si],
                device_id=peer,
                device_id_type=MESH,
            )

        quarters = (
            2 * my_x + my_y,
            2 * my_x + 1 - my_y,
            2 * (1 - my_x) + my_y,
            2 * (1 - my_x) + 1 - my_y,
        )

        def out_piece(c, k):
            out_ref[pl.ds(quarters[k] * Tq + c * Tc, Tc), :] = qbuf[
                c, k, :, :
            ].astype(jnp.float32)

        def tail_start(c):
            @pl.when(my_z == 0)
            def _():
                zcopy(c, 0, 2, 1, (my_x, my_y, my_z)).wait_recv()
                qbuf[c, 0, :, :] = zbuf[c, 0, :, :] + zbuf[c, 2, :, :]

            @pl.when(my_z == N_Z - 1)
            def _():
                zcopy(c, 0, 1, 0, (my_x, my_y, my_z)).wait_recv()
                qbuf[c, 0, :, :] = zbuf[c, 0, :, :] + zbuf[c, 1, :, :]

            @pl.when((my_z > 0) & (my_z < N_Z - 1))
            def _():
                qbuf[c, 0, :, :] = (
                    zbuf[c, 0, :, :] + zbuf[c, 1, :, :] + zbuf[c, 2, :, :]
                )

            out_piece(c, 0)
            qcopy(c, 0, 1, ysend, yrecv, 0, y_peer).start()
            qcopy(c, 0, 2, xsend, xrecv, 0, x_peer).start()

            @pl.when(my_z == 0)
            def _():
                zcopy(c, 0, 1, 0, (my_x, my_y, my_z)).wait_send()

            @pl.when((my_z > 0) & (my_z < N_Z - 1))
            def _():
                zcopy(c, 3, 1, 0, (my_x, my_y, my_z)).wait_send()
                zcopy(c, 4, 2, 1, (my_x, my_y, my_z)).wait_send()

            @pl.when(my_z == N_Z - 1)
            def _():
                zcopy(c, 0, 2, 1, (my_x, my_y, my_z)).wait_send()

        def diag_forward(c):
            if c % 2 == 0:
                qcopy(c, 0, 1, ysend, yrecv, 0, y_peer).wait_recv()
                qcopy(c, 1, 3, xsend, xrecv, 1, x_peer).start()
                out_piece(c, 1)
            else:
                qcopy(c, 0, 2, xsend, xrecv, 0, x_peer).wait_recv()
                qcopy(c, 2, 3, ysend, yrecv, 1, y_peer).start()
                out_piece(c, 2)

        def tail_finish(c):
            if c % 2 == 0:
                qcopy(c, 0, 2, xsend, xrecv, 0, x_peer).wait_recv()
                out_piece(c, 2)
                qcopy(c, 1, 3, xsend, xrecv, 1, x_peer).wait_recv()
                out_piece(c, 3)
                qcopy(c, 0, 0, ysend, yrecv, 0, y_peer).wait_send()
                qcopy(c, 0, 0, xsend, xrecv, 0, x_peer).wait_send()
                qcopy(c, 1, 0, xsend, xrecv, 1, x_peer).wait_send()
            else:
                qcopy(c, 0, 1, ysend, yrecv, 0, y_peer).wait_recv()
                out_piece(c, 1)
                qcopy(c, 2, 3, ysend, yrecv, 1, y_peer).wait_recv()
                out_piece(c, 3)
                qcopy(c, 0, 0, ysend, yrecv, 0, y_peer).wait_send()
                qcopy(c, 0, 0, xsend, xrecv, 0, x_peer).wait_send()
                qcopy(c, 2, 0, ysend, yrecv, 1, y_peer).wait_send()

        cnt0 = gather_issue(0)
        pl.semaphore_wait(barrier_sem, 4)
        gather_finish(0, cnt0)
        z_edge(0)
        cnt1 = gather_issue(1)
        gather_finish(1, cnt1)
        z_edge(1)
        z_mid(0)
        cnt2 = gather_issue(2)
        gather_finish(2, cnt2)
        z_edge(2)
        z_mid(1)
        cnt3 = gather_issue(3)
        gather_finish(3, cnt3)
        z_edge(3)
        z_mid(2)
        tail_start(0)
        z_mid(3)
        tail_start(1)
        diag_forward(0)
        tail_start(2)
        diag_forward(1)
        tail_start(3)
        diag_forward(2)
        diag_forward(3)
        tail_finish(0)
        tail_finish(1)
        tail_finish(2)
        tail_finish(3)

    grid_spec = pltpu.PrefetchScalarGridSpec(
        num_scalar_prefetch=1,
        in_specs=[
            pl.BlockSpec(memory_space=pl.ANY),
            pl.BlockSpec(memory_space=pltpu.VMEM),
        ],
        out_specs=pl.BlockSpec(memory_space=pltpu.VMEM),
        scratch_shapes=[
            pltpu.VMEM((CH, Tc, D), jnp.float32),
            pltpu.SemaphoreType.DMA((CH,)),
            pltpu.VMEM((CH, NSLOT, Tc, D), jnp.bfloat16),
            pltpu.VMEM((CH, 4, Tc, D), jnp.bfloat16),
            pltpu.SemaphoreType.DMA((CH, 2)),
            pltpu.SemaphoreType.DMA((CH, 2)),
            pltpu.SemaphoreType.DMA((CH, 2)),
            pltpu.SemaphoreType.DMA((CH, 2)),
            pltpu.SemaphoreType.DMA((CH, 2)),
            pltpu.SemaphoreType.DMA((CH, 2)),
        ],
    )

    return pl.pallas_call(
        body,
        out_shape=jax.ShapeDtypeStruct((T, D), jnp.float32),
        grid_spec=grid_spec,
        compiler_params=pltpu.CompilerParams(collective_id=0),
    )(ids32, E, ids2d)
